# baseline (speedup 1.0000x reference)
import jax
import jax.numpy as jnp
from jax.experimental import pallas as pl
from jax.experimental.pallas import tpu as pltpu
from jax.experimental.pallas import tpu_sc as plsc

_B = 4096
_DENSE = 13
_NF = 26
_V = 100000
_E = 32
_DC = _NF * _E
_INTER = _DC + _E
_PROJ = 512

_GATHER_WINDOW = 128
_NIDX = _B * _NF

_BB = 512
_GRID = _B // _BB


def _sc_gather(tables_flat, flat_idx):
    mesh = plsc.VectorSubcoreMesh(core_axis_name="core",
                                  subcore_axis_name="subcore")

    @pl.kernel(
        out_type=jax.ShapeDtypeStruct((_NIDX, _E), jnp.float32),
        mesh=mesh,
        compiler_params=pltpu.CompilerParams(use_tc_tiling_on_sc=False),
    )
    def gather_kernel(tab_hbm, idx_hbm, out_hbm):
        def body(i_vmem, o_vmem):
            pltpu.sync_copy(tab_hbm.at[i_vmem.at[0]], o_vmem)

        pltpu.emit_pipeline(
            body,
            grid=(_NIDX // _GATHER_WINDOW,),
            in_specs=[pl.BlockSpec((1, _GATHER_WINDOW),
                                   index_map=lambda i: (0, i))],
            out_specs=[pl.BlockSpec((_GATHER_WINDOW, _E),
                                    index_map=lambda i: (i, 0))],
            core_axis_name=("core", "subcore"),
            dimension_semantics=(pltpu.PARALLEL,),
        )(idx_hbm, out_hbm)

    return gather_kernel(tables_flat, flat_idx)


def _mm(a, b):
    return jnp.dot(a.astype(jnp.bfloat16), b,
                   preferred_element_type=jnp.float32)


def _dense_body(d_ref, emb_ref,
                b0_ref, bb0_ref, b1_ref, bb1_ref, b2_ref, bb2_ref,
                u0a_ref, u0b_ref, v0a_ref, v0b_ref, c0a_ref, c0b_ref,
                u1a_ref, u1b_ref, v1a_ref, v1b_ref, c1a_ref, c1b_ref,
                u2a_ref, u2b_ref, v2a_ref, v2b_ref, c2a_ref, c2b_ref,
                t0a_ref, t0b_ref, tb0_ref, t1_ref, tb1_ref,
                t2_ref, tb2_ref, t3_ref, tb3_ref, t4_ref, tb4_ref,
                o_ref):
    x = d_ref[...]
    x = jax.nn.relu(_mm(x, b0_ref[...]) + bb0_ref[...])
    x = jax.nn.relu(_mm(x, b1_ref[...]) + bb1_ref[...])
    xa = jax.nn.relu(_mm(x, b2_ref[...]) + bb2_ref[...])
    xb = emb_ref[...]

    la, lb = xa, xb
    cross = ((u0a_ref, u0b_ref, v0a_ref, v0b_ref, c0a_ref, c0b_ref),
             (u1a_ref, u1b_ref, v1a_ref, v1b_ref, c1a_ref, c1b_ref),
             (u2a_ref, u2b_ref, v2a_ref, v2b_ref, c2a_ref, c2b_ref))
    for ua, ub, va, vb, ca, cb in cross:
        p = (_mm(la, ua[...]) + _mm(lb, ub[...])).astype(jnp.bfloat16)
        vouta = jnp.dot(p, va[...], preferred_element_type=jnp.float32) + ca[...]
        voutb = jnp.dot(p, vb[...], preferred_element_type=jnp.float32) + cb[...]
        la = xa * vouta + la
        lb = xb * voutb + lb

    y = jax.nn.relu(_mm(la, t0a_ref[...]) + _mm(lb, t0b_ref[...]) + tb0_ref[...])
    y = jax.nn.relu(_mm(y, t1_ref[...]) + tb1_ref[...])
    y = jax.nn.relu(_mm(y, t2_ref[...]) + tb2_ref[...])
    y = jax.nn.relu(_mm(y, t3_ref[...]) + tb3_ref[...])
    o_ref[...] = jax.nn.sigmoid(_mm(y, t4_ref[...]) + tb4_ref[...])


def _full_spec(shape):
    nd = len(shape)
    return pl.BlockSpec(shape, lambda i, _nd=nd: (0,) * _nd)


def kernel(dense_features, embedding_lookups, tables,
           bW0, bB0, bW1, bB1, bW2, bB2,
           u0, v0, cb0, u1, v1, cb1, u2, v2, cb2,
           tW0, tB0, tW1, tB1, tW2, tB2, tW3, tB3, tW4, tB4):
    tables_flat = tables.reshape(_NF * _V, _E)
    flat_idx = (embedding_lookups
                + (jnp.arange(_NF, dtype=jnp.int32) * _V)[None, :])
    flat_idx = flat_idx.reshape(1, _NIDX)
    emb = _sc_gather(tables_flat, flat_idx).reshape(_B, _DC)

    bf = jnp.bfloat16
    w = {}
    w["b0"], w["b1"], w["b2"] = bW0.astype(bf), bW1.astype(bf), bW2.astype(bf)
    for i, (u, v, cb) in enumerate(((u0, v0, cb0), (u1, v1, cb1), (u2, v2, cb2))):
        w["u%da" % i] = u[:_E].astype(bf)
        w["u%db" % i] = u[_E:].astype(bf)
        w["v%da" % i] = v[:, :_E].astype(bf)
        w["v%db" % i] = v[:, _E:].astype(bf)
        w["c%da" % i] = cb[None, :_E]
        w["c%db" % i] = cb[None, _E:]
    w["t0a"], w["t0b"] = tW0[:_E].astype(bf), tW0[_E:].astype(bf)
    w["t1"], w["t2"], w["t3"], w["t4"] = (tW1.astype(bf), tW2.astype(bf),
                                          tW3.astype(bf), tW4.astype(bf))
    biases = dict(bb0=bB0[None, :], bb1=bB1[None, :], bb2=bB2[None, :],
                  tb0=tB0[None, :], tb1=tB1[None, :], tb2=tB2[None, :],
                  tb3=tB3[None, :], tb4=tB4[None, :])

    order = ["b0", "bb0", "b1", "bb1", "b2", "bb2",
             "u0a", "u0b", "v0a", "v0b", "c0a", "c0b",
             "u1a", "u1b", "v1a", "v1b", "c1a", "c1b",
             "u2a", "u2b", "v2a", "v2b", "c2a", "c2b",
             "t0a", "t0b", "tb0", "t1", "tb1",
             "t2", "tb2", "t3", "tb3", "t4", "tb4"]
    wargs = [w[k] if k in w else biases[k] for k in order]

    out = pl.pallas_call(
        _dense_body,
        grid=(_GRID,),
        in_specs=[
            pl.BlockSpec((_BB, _DENSE), lambda i: (i, 0)),
            pl.BlockSpec((_BB, _DC), lambda i: (i, 0)),
        ] + [_full_spec(a.shape) for a in wargs],
        out_specs=pl.BlockSpec((_BB, 1), lambda i: (i, 0)),
        out_shape=jax.ShapeDtypeStruct((_B, 1), jnp.float32),
    )(dense_features, emb, *wargs)

    return out.reshape(-1)


# speedup vs baseline: 1.3349x; 1.3349x over previous
import jax
import jax.numpy as jnp
from jax.experimental import pallas as pl
from jax.experimental.pallas import tpu as pltpu
from jax.experimental.pallas import tpu_sc as plsc

_B = 4096
_DENSE = 13
_NF = 26
_V = 100000
_E = 32
_DC = _NF * _E
_INTER = _DC + _E
_PROJ = 512

_GATHER_WINDOW = 128
_NIDX = _B * _NF

_BB = 512
_GRID = _B // _BB


_TVB = 2048
_TROWS = _TVB // 4
_NVB = -(-_V // _TVB)


def _transpose_body(in_ref, out_ref):
    x = in_ref[0]
    parts = [x[:, s * _TROWS:(s + 1) * _TROWS] for s in range(4)]
    out_ref[0] = jnp.concatenate(parts, axis=0).T


def _tc_transpose(tables_t):
    return pl.pallas_call(
        _transpose_body,
        grid=(_NF, _NVB),
        in_specs=[pl.BlockSpec((1, _E, _TVB), lambda f, j: (f, 0, j))],
        out_specs=pl.BlockSpec((1, _TROWS, 4 * _E), lambda f, j: (f, j, 0)),
        out_shape=jax.ShapeDtypeStruct((_NF, _V // 4, 4 * _E), jnp.float32),
    )(tables_t)


def _sc_gather(tables_flat, flat_idx):
    mesh = plsc.VectorSubcoreMesh(core_axis_name="core",
                                  subcore_axis_name="subcore")

    @pl.kernel(
        out_type=jax.ShapeDtypeStruct((_NIDX, _E), jnp.float32),
        mesh=mesh,
        compiler_params=pltpu.CompilerParams(use_tc_tiling_on_sc=False),
    )
    def gather_kernel(tab_hbm, idx_hbm, out_hbm):
        def body(i_vmem, o_vmem):
            pltpu.sync_copy(tab_hbm.at[i_vmem.at[0]], o_vmem)

        pltpu.emit_pipeline(
            body,
            grid=(_NIDX // _GATHER_WINDOW,),
            in_specs=[pl.BlockSpec((1, _GATHER_WINDOW),
                                   index_map=lambda i: (0, i))],
            out_specs=[pl.BlockSpec((_GATHER_WINDOW, _E),
                                    index_map=lambda i: (i, 0))],
            core_axis_name=("core", "subcore"),
            dimension_semantics=(pltpu.PARALLEL,),
        )(idx_hbm, out_hbm)

    return gather_kernel(tables_flat, flat_idx)


def _mm(a, b):
    return jnp.dot(a.astype(jnp.bfloat16), b,
                   preferred_element_type=jnp.float32)


def _dense_body(d_ref, emb_ref,
                b0_ref, bb0_ref, b1_ref, bb1_ref, b2_ref, bb2_ref,
                u0a_ref, u0b_ref, v0a_ref, v0b_ref, c0a_ref, c0b_ref,
                u1a_ref, u1b_ref, v1a_ref, v1b_ref, c1a_ref, c1b_ref,
                u2a_ref, u2b_ref, v2a_ref, v2b_ref, c2a_ref, c2b_ref,
                t0a_ref, t0b_ref, tb0_ref, t1_ref, tb1_ref,
                t2_ref, tb2_ref, t3_ref, tb3_ref, t4_ref, tb4_ref,
                o_ref):
    x = d_ref[...]
    x = jax.nn.relu(_mm(x, b0_ref[...]) + bb0_ref[...])
    x = jax.nn.relu(_mm(x, b1_ref[...]) + bb1_ref[...])
    xa = jax.nn.relu(_mm(x, b2_ref[...]) + bb2_ref[...])
    xb = emb_ref[...]

    la, lb = xa, xb
    cross = ((u0a_ref, u0b_ref, v0a_ref, v0b_ref, c0a_ref, c0b_ref),
             (u1a_ref, u1b_ref, v1a_ref, v1b_ref, c1a_ref, c1b_ref),
             (u2a_ref, u2b_ref, v2a_ref, v2b_ref, c2a_ref, c2b_ref))
    for ua, ub, va, vb, ca, cb in cross:
        p = (_mm(la, ua[...]) + _mm(lb, ub[...])).astype(jnp.bfloat16)
        vouta = jnp.dot(p, va[...], preferred_element_type=jnp.float32) + ca[...]
        voutb = jnp.dot(p, vb[...], preferred_element_type=jnp.float32) + cb[...]
        la = xa * vouta + la
        lb = xb * voutb + lb

    y = jax.nn.relu(_mm(la, t0a_ref[...]) + _mm(lb, t0b_ref[...]) + tb0_ref[...])
    y = jax.nn.relu(_mm(y, t1_ref[...]) + tb1_ref[...])
    y = jax.nn.relu(_mm(y, t2_ref[...]) + tb2_ref[...])
    y = jax.nn.relu(_mm(y, t3_ref[...]) + tb3_ref[...])
    o_ref[...] = jax.nn.sigmoid(_mm(y, t4_ref[...]) + tb4_ref[...])


def _full_spec(shape):
    nd = len(shape)
    return pl.BlockSpec(shape, lambda i, _nd=nd: (0,) * _nd)


def kernel(dense_features, embedding_lookups, tables,
           bW0, bB0, bW1, bB1, bW2, bB2,
           u0, v0, cb0, u1, v1, cb1, u2, v2, cb2,
           tW0, tB0, tW1, tB1, tW2, tB2, tW3, tB3, tW4, tB4):
    tables_t = jnp.transpose(tables, (0, 2, 1))
    tables_flat = _tc_transpose(tables_t).reshape(_NF * _V, _E)
    v = embedding_lookups
    flat_idx = ((v // _TVB) * _TVB + 4 * (v % _TROWS) + (v % _TVB) // _TROWS
                + (jnp.arange(_NF, dtype=jnp.int32) * _V)[None, :])
    flat_idx = flat_idx.reshape(1, _NIDX)
    emb = _sc_gather(tables_flat, flat_idx).reshape(_B, _DC)

    bf = jnp.bfloat16
    w = {}
    w["b0"], w["b1"], w["b2"] = bW0.astype(bf), bW1.astype(bf), bW2.astype(bf)
    for i, (u, v, cb) in enumerate(((u0, v0, cb0), (u1, v1, cb1), (u2, v2, cb2))):
        w["u%da" % i] = u[:_E].astype(bf)
        w["u%db" % i] = u[_E:].astype(bf)
        w["v%da" % i] = v[:, :_E].astype(bf)
        w["v%db" % i] = v[:, _E:].astype(bf)
        w["c%da" % i] = cb[None, :_E]
        w["c%db" % i] = cb[None, _E:]
    w["t0a"], w["t0b"] = tW0[:_E].astype(bf), tW0[_E:].astype(bf)
    w["t1"], w["t2"], w["t3"], w["t4"] = (tW1.astype(bf), tW2.astype(bf),
                                          tW3.astype(bf), tW4.astype(bf))
    biases = dict(bb0=bB0[None, :], bb1=bB1[None, :], bb2=bB2[None, :],
                  tb0=tB0[None, :], tb1=tB1[None, :], tb2=tB2[None, :],
                  tb3=tB3[None, :], tb4=tB4[None, :])

    order = ["b0", "bb0", "b1", "bb1", "b2", "bb2",
             "u0a", "u0b", "v0a", "v0b", "c0a", "c0b",
             "u1a", "u1b", "v1a", "v1b", "c1a", "c1b",
             "u2a", "u2b", "v2a", "v2b", "c2a", "c2b",
             "t0a", "t0b", "tb0", "t1", "tb1",
             "t2", "tb2", "t3", "tb3", "t4", "tb4"]
    wargs = [w[k] if k in w else biases[k] for k in order]

    out = pl.pallas_call(
        _dense_body,
        grid=(_GRID,),
        in_specs=[
            pl.BlockSpec((_BB, _DENSE), lambda i: (i, 0)),
            pl.BlockSpec((_BB, _DC), lambda i: (i, 0)),
        ] + [_full_spec(a.shape) for a in wargs],
        out_specs=pl.BlockSpec((_BB, 1), lambda i: (i, 0)),
        out_shape=jax.ShapeDtypeStruct((_B, 1), jnp.float32),
    )(dense_features, emb, *wargs)

    return out.reshape(-1)
